# baseline (speedup 1.0000x reference)
import jax
import jax.numpy as jnp
from jax import lax
from jax.experimental import pallas as pl
from jax.experimental.pallas import tpu as pltpu
from jax.experimental.pallas import tpu_sc as plsc

NUM_CLASSES = 1000
B, C = 4096, 26
CPAD = 32
NW = 32
PPW = B // NW
NBUF = 3


def _sc_body(xpad_hbm, zeros_hbm, out_hbm, idx_v, buf0, buf1, buf2, sem0, sem1, sem2):
    wid = lax.axis_index("s") * 2 + lax.axis_index("c")
    base = wid * PPW
    bufs = (buf0, buf1, buf2)
    sems = (sem0, sem1, sem2)

    pltpu.sync_copy(xpad_hbm.at[pl.ds(base * CPAD, PPW * CPAD)], idx_v)
    for buf in bufs:
        pltpu.sync_copy(zeros_hbm, buf)

    iot = lax.iota(jnp.int32, 16)
    ones16 = jnp.full((16,), 1, jnp.int32)
    zeros16 = jnp.full((16,), 0, jnp.int32)
    mask2 = (iot + 16) < C
    zer16i = jnp.full((16,), 0, jnp.int32)

    def scatter_plane(buf, p_local, val):
        off = p_local * CPAD
        i0 = idx_v[pl.ds(off, 16)]
        plsc.store_scatter(buf, [zer16i, iot, i0], val)
        i1 = idx_v[pl.ds(off + 16, 16)]
        plsc.store_scatter(buf, [zer16i, iot + 16, i1], val, mask=mask2)

    def process(buf, sem, p_local):
        @pl.when(p_local >= NBUF)
        def _():
            pltpu.make_async_copy(
                buf, out_hbm.at[pl.ds(base, 1)], sem
            ).wait()
            scatter_plane(buf, p_local - NBUF, zeros16)

        scatter_plane(buf, p_local, ones16)
        pltpu.make_async_copy(
            buf, out_hbm.at[pl.ds(base + p_local, 1)], sem
        ).start()

    def step(pp, carry):
        for k in range(NBUF):
            process(bufs[k], sems[k], NBUF * pp + k)
        return carry

    nfull = PPW // NBUF
    lax.fori_loop(0, nfull, step, 0)
    for k in range(PPW - nfull * NBUF):
        process(bufs[k], sems[k], nfull * NBUF + k)

    for buf, sem in zip(bufs, sems):
        pltpu.make_async_copy(buf, out_hbm.at[pl.ds(base, 1)], sem).wait()


def kernel(x1):
    xpad = jnp.pad(x1, ((0, 0), (0, CPAD - C))).reshape(-1)
    zeros = jnp.zeros((1, C, NUM_CLASSES), jnp.int32)
    mesh = plsc.VectorSubcoreMesh(core_axis_name="c", subcore_axis_name="s")
    run = pl.kernel(
        _sc_body,
        out_type=jax.ShapeDtypeStruct((B, C, NUM_CLASSES), jnp.int32),
        mesh=mesh,
        scratch_types=[
            pltpu.VMEM((PPW * CPAD,), jnp.int32),
            pltpu.VMEM((1, C, NUM_CLASSES), jnp.int32),
            pltpu.VMEM((1, C, NUM_CLASSES), jnp.int32),
            pltpu.VMEM((1, C, NUM_CLASSES), jnp.int32),
            pltpu.SemaphoreType.DMA,
            pltpu.SemaphoreType.DMA,
            pltpu.SemaphoreType.DMA,
        ],
        compiler_params=pltpu.CompilerParams(
            use_tc_tiling_on_sc=True, needs_layout_passes=False
        ),
    )
    return run(xpad, zeros)


# speedup vs baseline: 1.0133x; 1.0133x over previous
import jax
import jax.numpy as jnp
from jax import lax
from jax.experimental import pallas as pl
from jax.experimental.pallas import tpu as pltpu
from jax.experimental.pallas import tpu_sc as plsc

NUM_CLASSES = 1000
B, C = 4096, 26
CPAD = 32
NW = 32
PPW = B // NW


def _sc_body(xpad_hbm, zeros_hbm, out_hbm, idx_v, buf0, buf1, sem0, sem1):
    wid = lax.axis_index("s") * 2 + lax.axis_index("c")
    base = wid * PPW

    pltpu.sync_copy(xpad_hbm.at[pl.ds(base * CPAD, PPW * CPAD)], idx_v)
    pltpu.sync_copy(zeros_hbm, buf0)
    pltpu.sync_copy(zeros_hbm, buf1)

    iot = lax.iota(jnp.int32, 16)
    ones16 = jnp.full((16,), 1, jnp.int32)
    zeros16 = jnp.full((16,), 0, jnp.int32)
    mask2 = (iot + 16) < C
    zer16i = jnp.full((16,), 0, jnp.int32)

    def scatter_plane(buf, p_local, val):
        off = p_local * CPAD
        i0 = idx_v[pl.ds(off, 16)]
        plsc.store_scatter(buf, [zer16i, iot, i0], val)
        i1 = idx_v[pl.ds(off + 16, 16)]
        plsc.store_scatter(buf, [zer16i, iot + 16, i1], val, mask=mask2)

    def process(buf, sem, p_local):
        @pl.when(p_local >= 2)
        def _():
            pltpu.make_async_copy(
                buf,
                out_hbm.at[pl.ds(base, 1)],
                sem,
            ).wait()
            scatter_plane(buf, p_local - 2, zeros16)

        scatter_plane(buf, p_local, ones16)
        pltpu.make_async_copy(
            buf,
            out_hbm.at[pl.ds(base + p_local, 1)],
            sem,
        ).start()

    def step(pp, carry):
        process(buf0, sem0, 2 * pp)
        process(buf1, sem1, 2 * pp + 1)
        return carry

    lax.fori_loop(0, PPW // 2, step, 0)

    for buf, sem in ((buf0, sem0), (buf1, sem1)):
        pltpu.make_async_copy(
            buf,
            out_hbm.at[pl.ds(base, 1)],
            sem,
        ).wait()


def kernel(x1):
    xpad = jnp.pad(x1, ((0, 0), (0, CPAD - C))).reshape(-1)
    zeros = jnp.zeros((1, C, NUM_CLASSES), jnp.int32)
    mesh = plsc.VectorSubcoreMesh(core_axis_name="c", subcore_axis_name="s")
    run = pl.kernel(
        _sc_body,
        out_type=jax.ShapeDtypeStruct((B, C, NUM_CLASSES), jnp.int32),
        mesh=mesh,
        scratch_types=[
            pltpu.VMEM((PPW * CPAD,), jnp.int32),
            pltpu.VMEM((1, C, NUM_CLASSES), jnp.int32),
            pltpu.VMEM((1, C, NUM_CLASSES), jnp.int32),
            pltpu.SemaphoreType.DMA,
            pltpu.SemaphoreType.DMA,
        ],
        compiler_params=pltpu.CompilerParams(use_tc_tiling_on_sc=True, needs_layout_passes=False),
    )
    return run(xpad, zeros)
